# Initial kernel scaffold; baseline (speedup 1.0000x reference)
import jax
import jax.numpy as jnp
from jax.experimental import pallas as pl


def kernel(x, edge_index, W0, b0, W1, b1):
    raise NotImplementedError("write your pallas kernel here")


# speedup vs baseline: 17.2321x; 17.2321x over previous
import functools

import jax
import jax.numpy as jnp
from jax import lax
from jax.experimental import pallas as pl
from jax.experimental.pallas import tpu as pltpu
from jax.experimental.pallas import tpu_sc as plsc

_N = 10000
_D = 128
_E = 320000

_NC = 2
_NS = 16
_NW = _NC * _NS
_EPW = _E // _NW
_K = 80
_CH = _EPW // _K
_NR = 10240
_RPT = _NR // _NS
_ZR = 128

_NP = 10240
_DW = 128
_DPT = _NP // _NS

_mesh = plsc.VectorSubcoreMesh(
    core_axis_name="c", subcore_axis_name="s", num_cores=_NC, num_subcores=_NS
)


@functools.partial(
    pl.kernel,
    out_type=jax.ShapeDtypeStruct((_NC, _NP, _DW), jnp.float32),
    mesh=_mesh,
    scratch_types=[
        pltpu.VMEM_SHARED((_NP, _DW), jnp.float32),
        pltpu.VMEM((_CH, _K), jnp.int32),
        pltpu.VMEM((_K, _DW), jnp.float32),
    ],
)
def _deg_kernel(dst_hbm, ones_hbm, zeros_hbm, out_hbm, acc, didx, ones_v):
    cc = lax.axis_index("c")
    sid = lax.axis_index("s")
    wid = sid * _NC + cc
    pltpu.sync_copy(ones_hbm, ones_v)
    pltpu.sync_copy(dst_hbm.at[wid], didx)
    pltpu.sync_copy(zeros_hbm, acc.at[pl.ds(sid * _DPT, _DPT)])
    plsc.subcore_barrier()

    def body(ci, carry):
        pltpu.sync_copy(ones_v, acc.at[didx.at[ci]], add=True)
        return carry

    lax.fori_loop(0, _CH, body, 0)
    plsc.subcore_barrier()
    pltpu.sync_copy(
        acc.at[pl.ds(sid * _DPT, _DPT)],
        out_hbm.at[cc, pl.ds(sid * _DPT, _DPT)],
    )


@functools.partial(
    pl.kernel,
    out_type=jax.ShapeDtypeStruct((_NC, _NR, _D), jnp.float32),
    mesh=_mesh,
    scratch_types=[
        pltpu.VMEM_SHARED((_NR, _D), jnp.float32),
        pltpu.VMEM((_CH, _K), jnp.int32),
        pltpu.VMEM((_CH, _K), jnp.int32),
        pltpu.VMEM((_K, _D), jnp.float32),
        pltpu.SemaphoreType.DMA,
    ],
)
def _edge_kernel(src_hbm, dst_hbm, zeros_hbm, g_hbm, out_hbm,
                 acc, sidx, didx, rows_v, sem):
    cc = lax.axis_index("c")
    sid = lax.axis_index("s")
    wid = sid * _NC + cc
    pltpu.sync_copy(src_hbm.at[wid], sidx)
    pltpu.sync_copy(dst_hbm.at[wid], didx)
    pltpu.sync_copy(zeros_hbm, acc.at[pl.ds(sid * _RPT, _RPT)])
    plsc.subcore_barrier()

    def body(ci, carry):
        pltpu.async_copy(g_hbm.at[sidx.at[ci]], rows_v, sem).wait()
        pltpu.sync_copy(rows_v, acc.at[didx.at[ci]], add=True)
        return carry

    lax.fori_loop(0, _CH, body, 0)
    plsc.subcore_barrier()
    pltpu.sync_copy(
        acc.at[pl.ds(sid * _RPT, _RPT)],
        out_hbm.at[cc, pl.ds(sid * _RPT, _RPT)],
    )


def _mm1_body(x_ref, w_ref, d0_ref, d1_ref, out_ref):
    dinv = lax.rsqrt(d0_ref[...] + d1_ref[...] + 1.0)
    hw = jnp.dot(x_ref[...], w_ref[...], preferred_element_type=jnp.float32)
    out_ref[...] = hw * dinv


def _mid_body(sa_ref, sb_ref, g_ref, d0_ref, d1_ref, b_ref, w_ref, out_ref):
    dinv = lax.rsqrt(d0_ref[...] + d1_ref[...] + 1.0)
    h = (sa_ref[...] + sb_ref[...] + g_ref[...]) * dinv + b_ref[...]
    h = jnp.maximum(h, 0.0)
    hw = jnp.dot(h, w_ref[...], preferred_element_type=jnp.float32)
    out_ref[...] = hw * dinv


def _final_body(sa_ref, sb_ref, g_ref, d0_ref, d1_ref, b_ref, out_ref):
    dinv = lax.rsqrt(d0_ref[...] + d1_ref[...] + 1.0)
    h = (sa_ref[...] + sb_ref[...] + g_ref[...]) * dinv + b_ref[...]
    h = jnp.maximum(h, 0.0)
    nrm = jnp.sqrt(jnp.sum(h * h, axis=1, keepdims=True))
    out_ref[...] = h / (nrm + 1e-12)


_f32_out = jax.ShapeDtypeStruct((_N, _D), jnp.float32)
_mm1 = pl.pallas_call(_mm1_body, out_shape=_f32_out)
_mid = pl.pallas_call(_mid_body, out_shape=_f32_out)
_final = pl.pallas_call(_final_body, out_shape=_f32_out)


@jax.jit
def _impl(x, edge_index, W0, b0, W1, b1):
    src = edge_index[0].astype(jnp.int32).reshape(_NW, _CH, _K)
    dst = edge_index[1].astype(jnp.int32).reshape(_NW, _CH, _K)
    ones = jnp.ones((_K, _DW), jnp.float32)
    dzeros = jnp.zeros((_DPT, _DW), jnp.float32)
    rzeros = jnp.zeros((_RPT, _D), jnp.float32)

    degp = _deg_kernel(dst, ones, dzeros)
    d0 = degp[0, :_N, 0].reshape(_N, 1)
    d1 = degp[1, :_N, 0].reshape(_N, 1)
    b0r = b0.reshape(1, _D)
    b1r = b1.reshape(1, _D)

    g0 = _mm1(x, W0, d0, d1)
    s0 = _edge_kernel(src, dst, rzeros, g0)
    g1 = _mid(s0[0, :_N], s0[1, :_N], g0, d0, d1, b0r, W1)
    s1 = _edge_kernel(src, dst, rzeros, g1)
    return _final(s1[0, :_N], s1[1, :_N], g1, d0, d1, b1r)


def kernel(x, edge_index, W0, b0, W1, b1):
    return _impl(x, edge_index, W0, b0, W1, b1)


# speedup vs baseline: 20.3575x; 1.1814x over previous
import functools

import jax
import jax.numpy as jnp
from jax import lax
from jax.experimental import pallas as pl
from jax.experimental.pallas import tpu as pltpu
from jax.experimental.pallas import tpu_sc as plsc

_N = 10000
_D = 128
_E = 320000

_NC = 2
_NS = 16
_NW = _NC * _NS
_EPW = _E // _NW
_K = 80
_CH = _EPW // _K
_NST = 5
_CHS = _CH // _NST
_NR = 10240
_RPT = _NR // _NS
_ZR = 128

_NP = 10240
_DW = 128
_DPT = _NP // _NS

_mesh = plsc.VectorSubcoreMesh(
    core_axis_name="c", subcore_axis_name="s", num_cores=_NC, num_subcores=_NS
)


@functools.partial(
    pl.kernel,
    out_type=jax.ShapeDtypeStruct((_NC, _NP, _DW), jnp.float32),
    mesh=_mesh,
    scratch_types=[
        pltpu.VMEM_SHARED((_NP, _DW), jnp.float32),
        pltpu.VMEM((_CH, _K), jnp.int32),
        pltpu.VMEM((_K, _DW), jnp.float32),
    ],
)
def _deg_kernel(dst_hbm, ones_hbm, zeros_hbm, out_hbm, acc, didx, ones_v):
    cc = lax.axis_index("c")
    sid = lax.axis_index("s")
    wid = sid * _NC + cc
    pltpu.sync_copy(ones_hbm, ones_v)
    pltpu.sync_copy(dst_hbm.at[wid], didx)
    pltpu.sync_copy(zeros_hbm, acc.at[pl.ds(sid * _DPT, _DPT)])
    plsc.subcore_barrier()

    def body(ci, carry):
        pltpu.sync_copy(ones_v, acc.at[didx.at[ci]], add=True)
        return carry

    lax.fori_loop(0, _CH, body, 0)
    plsc.subcore_barrier()
    pltpu.sync_copy(
        acc.at[pl.ds(sid * _DPT, _DPT)],
        out_hbm.at[cc, pl.ds(sid * _DPT, _DPT)],
    )


@functools.partial(
    pl.kernel,
    out_type=jax.ShapeDtypeStruct((_NC, _NR, _D), jnp.float32),
    mesh=_mesh,
    scratch_types=[
        pltpu.VMEM_SHARED((_NR, _D), jnp.float32),
        pltpu.VMEM((_CHS, _K), jnp.int32),
        pltpu.VMEM((_CHS, _K), jnp.int32),
        pltpu.VMEM((_K, _D), jnp.float32),
        pltpu.VMEM((_K, _D), jnp.float32),
        pltpu.SemaphoreType.DMA,
        pltpu.SemaphoreType.DMA,
    ],
)
def _edge_kernel(src_hbm, dst_hbm, zeros_hbm, g_hbm, out_hbm,
                 acc, sidx, didx, rows_a, rows_b, sem_a, sem_b):
    cc = lax.axis_index("c")
    sid = lax.axis_index("s")
    wid = sid * _NC + cc
    pltpu.sync_copy(zeros_hbm, acc.at[pl.ds(sid * _RPT, _RPT)])
    plsc.subcore_barrier()

    def stage(si, carry):
        pltpu.sync_copy(src_hbm.at[wid, si], sidx)
        pltpu.sync_copy(dst_hbm.at[wid, si], didx)
        pltpu.async_copy(g_hbm.at[sidx.at[0]], rows_a, sem_a)

        def body(p, carry2):
            i = 2 * p
            pltpu.make_async_copy(g_hbm.at[sidx.at[0]], rows_a, sem_a).wait()
            pltpu.async_copy(g_hbm.at[sidx.at[i + 1]], rows_b, sem_b)
            pltpu.sync_copy(rows_a, acc.at[didx.at[i]], add=True)
            pltpu.make_async_copy(g_hbm.at[sidx.at[0]], rows_b, sem_b).wait()
            pltpu.async_copy(g_hbm.at[sidx.at[i + 2]], rows_a, sem_a)
            pltpu.sync_copy(rows_b, acc.at[didx.at[i + 1]], add=True)
            return carry2

        lax.fori_loop(0, _CHS // 2, body, 0)
        pltpu.make_async_copy(g_hbm.at[sidx.at[0]], rows_a, sem_a).wait()
        pltpu.sync_copy(rows_a, acc.at[didx.at[_CHS - 1]], add=True)
        return carry

    lax.fori_loop(0, _NST, stage, 0)
    plsc.subcore_barrier()
    pltpu.sync_copy(
        acc.at[pl.ds(sid * _RPT, _RPT)],
        out_hbm.at[cc, pl.ds(sid * _RPT, _RPT)],
    )


def _mm1_body(x_ref, w_ref, d0_ref, d1_ref, out_ref):
    dinv = lax.rsqrt(d0_ref[...] + d1_ref[...] + 1.0)
    hw = jnp.dot(x_ref[...], w_ref[...], preferred_element_type=jnp.float32)
    out_ref[...] = hw * dinv


def _mid_body(sa_ref, sb_ref, g_ref, d0_ref, d1_ref, b_ref, w_ref, out_ref):
    dinv = lax.rsqrt(d0_ref[...] + d1_ref[...] + 1.0)
    h = (sa_ref[...] + sb_ref[...] + g_ref[...]) * dinv + b_ref[...]
    h = jnp.maximum(h, 0.0)
    hw = jnp.dot(h, w_ref[...], preferred_element_type=jnp.float32)
    out_ref[...] = hw * dinv


def _final_body(sa_ref, sb_ref, g_ref, d0_ref, d1_ref, b_ref, out_ref):
    dinv = lax.rsqrt(d0_ref[...] + d1_ref[...] + 1.0)
    h = (sa_ref[...] + sb_ref[...] + g_ref[...]) * dinv + b_ref[...]
    h = jnp.maximum(h, 0.0)
    nrm = jnp.sqrt(jnp.sum(h * h, axis=1, keepdims=True))
    out_ref[...] = h / (nrm + 1e-12)


_f32_out = jax.ShapeDtypeStruct((_N, _D), jnp.float32)
_mm1 = pl.pallas_call(_mm1_body, out_shape=_f32_out)
_mid = pl.pallas_call(_mid_body, out_shape=_f32_out)
_final = pl.pallas_call(_final_body, out_shape=_f32_out)


@jax.jit
def _impl(x, edge_index, W0, b0, W1, b1):
    src = edge_index[0].astype(jnp.int32)
    dst = edge_index[1].astype(jnp.int32)
    dst3 = dst.reshape(_NW, _CH, _K)
    src4 = src.reshape(_NW, _NST, _CHS, _K)
    dst4 = dst.reshape(_NW, _NST, _CHS, _K)
    ones = jnp.ones((_K, _DW), jnp.float32)
    dzeros = jnp.zeros((_DPT, _DW), jnp.float32)
    rzeros = jnp.zeros((_RPT, _D), jnp.float32)

    degp = _deg_kernel(dst3, ones, dzeros)
    d0 = degp[0, :_N, 0].reshape(_N, 1)
    d1 = degp[1, :_N, 0].reshape(_N, 1)
    b0r = b0.reshape(1, _D)
    b1r = b1.reshape(1, _D)

    g0 = _mm1(x, W0, d0, d1)
    s0 = _edge_kernel(src4, dst4, rzeros, g0)
    g1 = _mid(s0[0, :_N], s0[1, :_N], g0, d0, d1, b0r, W1)
    s1 = _edge_kernel(src4, dst4, rzeros, g1)
    return _final(s1[0, :_N], s1[1, :_N], g1, d0, d1, b1r)


def kernel(x, edge_index, W0, b0, W1, b1):
    return _impl(x, edge_index, W0, b0, W1, b1)


# speedup vs baseline: 21.1060x; 1.0368x over previous
import functools

import jax
import jax.numpy as jnp
from jax import lax
from jax.experimental import pallas as pl
from jax.experimental.pallas import tpu as pltpu
from jax.experimental.pallas import tpu_sc as plsc

_N = 10000
_D = 128
_E = 320000

_NC = 2
_NS = 16
_NW = _NC * _NS
_EPW = _E // _NW
_K = 80
_CH = _EPW // _K

_EK = 100
_ENST = 5
_ECS = _EPW // (_EK * _ENST)
_NR = 10240
_RPT = _NR // _NS
_ZR = 128

_NP = 10240
_DW = 128
_DPT = _NP // _NS

_mesh = plsc.VectorSubcoreMesh(
    core_axis_name="c", subcore_axis_name="s", num_cores=_NC, num_subcores=_NS
)


@functools.partial(
    pl.kernel,
    out_type=jax.ShapeDtypeStruct((_NC, _NP, _DW), jnp.float32),
    mesh=_mesh,
    scratch_types=[
        pltpu.VMEM_SHARED((_NP, _DW), jnp.float32),
        pltpu.VMEM((_CH, _K), jnp.int32),
        pltpu.VMEM((_K, _DW), jnp.float32),
    ],
)
def _deg_kernel(dst_hbm, ones_hbm, zeros_hbm, out_hbm, acc, didx, ones_v):
    cc = lax.axis_index("c")
    sid = lax.axis_index("s")
    wid = sid * _NC + cc
    pltpu.sync_copy(ones_hbm, ones_v)
    pltpu.sync_copy(dst_hbm.at[wid], didx)
    pltpu.sync_copy(zeros_hbm, acc.at[pl.ds(sid * _DPT, _DPT)])
    plsc.subcore_barrier()

    def body(ci, carry):
        pltpu.sync_copy(ones_v, acc.at[didx.at[ci]], add=True)
        return carry

    lax.fori_loop(0, _CH, body, 0)
    plsc.subcore_barrier()
    pltpu.sync_copy(
        acc.at[pl.ds(sid * _DPT, _DPT)],
        out_hbm.at[cc, pl.ds(sid * _DPT, _DPT)],
    )


@functools.partial(
    pl.kernel,
    out_type=jax.ShapeDtypeStruct((_NC, _NR, _D), jnp.float32),
    mesh=_mesh,
    scratch_types=[
        pltpu.VMEM_SHARED((_NR, _D), jnp.float32),
        pltpu.VMEM((_ECS, _EK), jnp.int32),
        pltpu.VMEM((_ECS, _EK), jnp.int32),
        pltpu.VMEM((_EK, _D), jnp.float32),
        pltpu.VMEM((_EK, _D), jnp.float32),
        pltpu.SemaphoreType.DMA,
        pltpu.SemaphoreType.DMA,
        pltpu.SemaphoreType.DMA,
        pltpu.SemaphoreType.DMA,
    ],
)
def _edge_kernel(src_hbm, dst_hbm, zeros_hbm, g_hbm, out_hbm,
                 acc, sidx, didx, rows_a, rows_b, sem_ga, sem_gb, sem_sa, sem_sb):
    cc = lax.axis_index("c")
    sid = lax.axis_index("s")
    wid = sid * _NC + cc
    pltpu.sync_copy(zeros_hbm, acc.at[pl.ds(sid * _RPT, _RPT)])
    plsc.subcore_barrier()

    def wait_g(sem, buf):
        pltpu.make_async_copy(g_hbm.at[sidx.at[0]], buf, sem).wait()

    def wait_s(sem, buf):
        pltpu.make_async_copy(buf, acc.at[didx.at[0]], sem).wait()

    def stage(si, carry):
        pltpu.sync_copy(src_hbm.at[wid, si], sidx)
        pltpu.sync_copy(dst_hbm.at[wid, si], didx)
        pltpu.async_copy(g_hbm.at[sidx.at[0]], rows_a, sem_ga)
        pltpu.async_copy(g_hbm.at[sidx.at[1]], rows_b, sem_gb)

        def body(p, carry2):
            i = 2 * p
            wait_g(sem_ga, rows_a)
            pltpu.async_copy(rows_a, acc.at[didx.at[i]], sem_sa, add=True)
            wait_g(sem_gb, rows_b)
            pltpu.async_copy(rows_b, acc.at[didx.at[i + 1]], sem_sb, add=True)
            wait_s(sem_sa, rows_a)
            pltpu.async_copy(g_hbm.at[sidx.at[i + 2]], rows_a, sem_ga)
            wait_s(sem_sb, rows_b)
            pltpu.async_copy(g_hbm.at[sidx.at[i + 3]], rows_b, sem_gb)
            return carry2

        lax.fori_loop(0, _ECS // 2 - 1, body, 0)
        wait_g(sem_ga, rows_a)
        pltpu.async_copy(rows_a, acc.at[didx.at[_ECS - 2]], sem_sa, add=True)
        wait_g(sem_gb, rows_b)
        pltpu.async_copy(rows_b, acc.at[didx.at[_ECS - 1]], sem_sb, add=True)
        wait_s(sem_sa, rows_a)
        wait_s(sem_sb, rows_b)
        return carry

    lax.fori_loop(0, _ENST, stage, 0)
    plsc.subcore_barrier()
    pltpu.sync_copy(
        acc.at[pl.ds(sid * _RPT, _RPT)],
        out_hbm.at[cc, pl.ds(sid * _RPT, _RPT)],
    )


def _mm1_body(x_ref, w_ref, d0_ref, d1_ref, out_ref):
    dinv = lax.rsqrt(d0_ref[...] + d1_ref[...] + 1.0)
    hw = jnp.dot(x_ref[...], w_ref[...], preferred_element_type=jnp.float32)
    out_ref[...] = hw * dinv


def _mid_body(sa_ref, sb_ref, g_ref, d0_ref, d1_ref, b_ref, w_ref, out_ref):
    dinv = lax.rsqrt(d0_ref[...] + d1_ref[...] + 1.0)
    h = (sa_ref[...] + sb_ref[...] + g_ref[...]) * dinv + b_ref[...]
    h = jnp.maximum(h, 0.0)
    hw = jnp.dot(h, w_ref[...], preferred_element_type=jnp.float32)
    out_ref[...] = hw * dinv


def _final_body(sa_ref, sb_ref, g_ref, d0_ref, d1_ref, b_ref, out_ref):
    dinv = lax.rsqrt(d0_ref[...] + d1_ref[...] + 1.0)
    h = (sa_ref[...] + sb_ref[...] + g_ref[...]) * dinv + b_ref[...]
    h = jnp.maximum(h, 0.0)
    nrm = jnp.sqrt(jnp.sum(h * h, axis=1, keepdims=True))
    out_ref[...] = h / (nrm + 1e-12)


_f32_out = jax.ShapeDtypeStruct((_N, _D), jnp.float32)
_mm1 = pl.pallas_call(_mm1_body, out_shape=_f32_out)
_mid = pl.pallas_call(_mid_body, out_shape=_f32_out)
_final = pl.pallas_call(_final_body, out_shape=_f32_out)


@jax.jit
def _impl(x, edge_index, W0, b0, W1, b1):
    src = edge_index[0].astype(jnp.int32)
    dst = edge_index[1].astype(jnp.int32)
    dst3 = dst.reshape(_NW, _CH, _K)
    src4 = src.reshape(_NW, _ENST, _ECS, _EK)
    dst4 = dst.reshape(_NW, _ENST, _ECS, _EK)
    ones = jnp.ones((_K, _DW), jnp.float32)
    dzeros = jnp.zeros((_DPT, _DW), jnp.float32)
    rzeros = jnp.zeros((_RPT, _D), jnp.float32)

    degp = _deg_kernel(dst3, ones, dzeros)
    d0 = degp[0, :_N, 0].reshape(_N, 1)
    d1 = degp[1, :_N, 0].reshape(_N, 1)
    b0r = b0.reshape(1, _D)
    b1r = b1.reshape(1, _D)

    g0 = _mm1(x, W0, d0, d1)
    s0 = _edge_kernel(src4, dst4, rzeros, g0)
    g1 = _mid(s0[0, :_N], s0[1, :_N], g0, d0, d1, b0r, W1)
    s1 = _edge_kernel(src4, dst4, rzeros, g1)
    return _final(s1[0, :_N], s1[1, :_N], g1, d0, d1, b1r)


def kernel(x, edge_index, W0, b0, W1, b1):
    return _impl(x, edge_index, W0, b0, W1, b1)


# speedup vs baseline: 21.1151x; 1.0004x over previous
import functools

import jax
import jax.numpy as jnp
from jax import lax
from jax.experimental import pallas as pl
from jax.experimental.pallas import tpu as pltpu
from jax.experimental.pallas import tpu_sc as plsc

_N = 10000
_D = 128
_E = 320000

_NC = 2
_NS = 16
_NW = _NC * _NS
_EPW = _E // _NW
_K = 80
_CH = _EPW // _K

_EK = 100
_ENST = 5
_ECS = _EPW // (_EK * _ENST)
_NR = 10240
_RPT = _NR // _NS
_ZR = 128

_NP = 10240
_DW = 128
_DPT = _NP // _NS

_mesh = plsc.VectorSubcoreMesh(
    core_axis_name="c", subcore_axis_name="s", num_cores=_NC, num_subcores=_NS
)


@functools.partial(
    pl.kernel,
    out_type=jax.ShapeDtypeStruct((_NC, _NP, _DW), jnp.float32),
    mesh=_mesh,
    scratch_types=[
        pltpu.VMEM_SHARED((_NP, _DW), jnp.float32),
        pltpu.VMEM((_CH, _K), jnp.int32),
        pltpu.VMEM((_K, _DW), jnp.float32),
        pltpu.SemaphoreType.DMA,
    ],
)
def _deg_kernel(dst_hbm, ones_hbm, zeros_hbm, out_hbm, acc, didx, ones_v, sem):
    cc = lax.axis_index("c")
    sid = lax.axis_index("s")
    wid = sid * _NC + cc
    pltpu.sync_copy(ones_hbm, ones_v)
    pltpu.sync_copy(dst_hbm.at[wid], didx)
    pltpu.sync_copy(zeros_hbm, acc.at[pl.ds(sid * _DPT, _DPT)])
    plsc.subcore_barrier()

    def block(bi, carry):
        def fire(ci, carry2):
            pltpu.async_copy(ones_v, acc.at[didx.at[bi * 25 + ci]], sem,
                             add=True)
            return carry2

        lax.fori_loop(0, 25, fire, 0)

        def drain(ci, carry2):
            pltpu.make_async_copy(ones_v, acc.at[didx.at[0]], sem).wait()
            return carry2

        lax.fori_loop(0, 25, drain, 0)
        return carry

    lax.fori_loop(0, _CH // 25, block, 0)
    plsc.subcore_barrier()
    pltpu.sync_copy(
        acc.at[pl.ds(sid * _DPT, _DPT)],
        out_hbm.at[cc, pl.ds(sid * _DPT, _DPT)],
    )


@functools.partial(
    pl.kernel,
    out_type=jax.ShapeDtypeStruct((_NC, _NR, _D), jnp.float32),
    mesh=_mesh,
    scratch_types=[
        pltpu.VMEM_SHARED((_NR, _D), jnp.float32),
        pltpu.VMEM((_ECS, _EK), jnp.int32),
        pltpu.VMEM((_ECS, _EK), jnp.int32),
        pltpu.VMEM((_EK, _D), jnp.float32),
        pltpu.VMEM((_EK, _D), jnp.float32),
        pltpu.SemaphoreType.DMA,
        pltpu.SemaphoreType.DMA,
        pltpu.SemaphoreType.DMA,
        pltpu.SemaphoreType.DMA,
    ],
)
def _edge_kernel(src_hbm, dst_hbm, zeros_hbm, g_hbm, out_hbm,
                 acc, sidx, didx, rows_a, rows_b, sem_ga, sem_gb, sem_sa, sem_sb):
    cc = lax.axis_index("c")
    sid = lax.axis_index("s")
    wid = sid * _NC + cc
    pltpu.sync_copy(zeros_hbm, acc.at[pl.ds(sid * _RPT, _RPT)])
    plsc.subcore_barrier()

    def wait_g(sem, buf):
        pltpu.make_async_copy(g_hbm.at[sidx.at[0]], buf, sem).wait()

    def wait_s(sem, buf):
        pltpu.make_async_copy(buf, acc.at[didx.at[0]], sem).wait()

    def stage(si, carry):
        pltpu.sync_copy(src_hbm.at[wid, si], sidx)
        pltpu.sync_copy(dst_hbm.at[wid, si], didx)
        pltpu.async_copy(g_hbm.at[sidx.at[0]], rows_a, sem_ga)
        pltpu.async_copy(g_hbm.at[sidx.at[1]], rows_b, sem_gb)

        def body(p, carry2):
            i = 2 * p
            wait_g(sem_ga, rows_a)
            pltpu.async_copy(rows_a, acc.at[didx.at[i]], sem_sa, add=True)
            wait_g(sem_gb, rows_b)
            pltpu.async_copy(rows_b, acc.at[didx.at[i + 1]], sem_sb, add=True)
            wait_s(sem_sa, rows_a)
            pltpu.async_copy(g_hbm.at[sidx.at[i + 2]], rows_a, sem_ga)
            wait_s(sem_sb, rows_b)
            pltpu.async_copy(g_hbm.at[sidx.at[i + 3]], rows_b, sem_gb)
            return carry2

        lax.fori_loop(0, _ECS // 2 - 1, body, 0)
        wait_g(sem_ga, rows_a)
        pltpu.async_copy(rows_a, acc.at[didx.at[_ECS - 2]], sem_sa, add=True)
        wait_g(sem_gb, rows_b)
        pltpu.async_copy(rows_b, acc.at[didx.at[_ECS - 1]], sem_sb, add=True)
        wait_s(sem_sa, rows_a)
        wait_s(sem_sb, rows_b)
        return carry

    lax.fori_loop(0, _ENST, stage, 0)
    plsc.subcore_barrier()
    pltpu.sync_copy(
        acc.at[pl.ds(sid * _RPT, _RPT)],
        out_hbm.at[cc, pl.ds(sid * _RPT, _RPT)],
    )


def _mm1_body(x_ref, w_ref, d0_ref, d1_ref, out_ref):
    dinv = lax.rsqrt(d0_ref[...] + d1_ref[...] + 1.0)
    hw = jnp.dot(x_ref[...], w_ref[...], preferred_element_type=jnp.float32)
    out_ref[...] = hw * dinv


def _mid_body(sa_ref, sb_ref, g_ref, d0_ref, d1_ref, b_ref, w_ref, out_ref):
    dinv = lax.rsqrt(d0_ref[...] + d1_ref[...] + 1.0)
    h = (sa_ref[...] + sb_ref[...] + g_ref[...]) * dinv + b_ref[...]
    h = jnp.maximum(h, 0.0)
    hw = jnp.dot(h, w_ref[...], preferred_element_type=jnp.float32)
    out_ref[...] = hw * dinv


def _final_body(sa_ref, sb_ref, g_ref, d0_ref, d1_ref, b_ref, out_ref):
    dinv = lax.rsqrt(d0_ref[...] + d1_ref[...] + 1.0)
    h = (sa_ref[...] + sb_ref[...] + g_ref[...]) * dinv + b_ref[...]
    h = jnp.maximum(h, 0.0)
    nrm = jnp.sqrt(jnp.sum(h * h, axis=1, keepdims=True))
    out_ref[...] = h / (nrm + 1e-12)


_f32_out = jax.ShapeDtypeStruct((_N, _D), jnp.float32)
_mm1 = pl.pallas_call(_mm1_body, out_shape=_f32_out)
_mid = pl.pallas_call(_mid_body, out_shape=_f32_out)
_final = pl.pallas_call(_final_body, out_shape=_f32_out)


@jax.jit
def _impl(x, edge_index, W0, b0, W1, b1):
    src = edge_index[0].astype(jnp.int32)
    dst = edge_index[1].astype(jnp.int32)
    dst3 = dst.reshape(_NW, _CH, _K)
    src4 = src.reshape(_NW, _ENST, _ECS, _EK)
    dst4 = dst.reshape(_NW, _ENST, _ECS, _EK)
    ones = jnp.ones((_K, _DW), jnp.float32)
    dzeros = jnp.zeros((_DPT, _DW), jnp.float32)
    rzeros = jnp.zeros((_RPT, _D), jnp.float32)

    degp = _deg_kernel(dst3, ones, dzeros)
    d0 = degp[0, :_N, 0].reshape(_N, 1)
    d1 = degp[1, :_N, 0].reshape(_N, 1)
    b0r = b0.reshape(1, _D)
    b1r = b1.reshape(1, _D)

    g0 = _mm1(x, W0, d0, d1)
    s0 = _edge_kernel(src4, dst4, rzeros, g0)
    g1 = _mid(s0[0, :_N], s0[1, :_N], g0, d0, d1, b0r, W1)
    s1 = _edge_kernel(src4, dst4, rzeros, g1)
    return _final(s1[0, :_N], s1[1, :_N], g1, d0, d1, b1r)


def kernel(x, edge_index, W0, b0, W1, b1):
    return _impl(x, edge_index, W0, b0, W1, b1)


# speedup vs baseline: 22.0836x; 1.0459x over previous
import functools

import jax
import jax.numpy as jnp
from jax import lax
from jax.experimental import pallas as pl
from jax.experimental.pallas import tpu as pltpu
from jax.experimental.pallas import tpu_sc as plsc

_N = 10000
_D = 128
_E = 320000

_NC = 2
_NS = 16
_NW = _NC * _NS
_EPW = _E // _NW
_K = 80
_CH = _EPW // _K

_EK = 100
_ENST = 5
_ECS = _EPW // (_EK * _ENST)
_NR = 10240
_RPT = _NR // _NS
_ZR = 128

_NP = 10240
_DW = 128
_DPT = _NP // _NS

_mesh = plsc.VectorSubcoreMesh(
    core_axis_name="c", subcore_axis_name="s", num_cores=_NC, num_subcores=_NS
)


@functools.partial(
    pl.kernel,
    out_type=jax.ShapeDtypeStruct((_NC, _NP, _DW), jnp.float32),
    mesh=_mesh,
    scratch_types=[
        pltpu.VMEM_SHARED((_NP, _DW), jnp.float32),
        pltpu.VMEM((_CH, _K), jnp.int32),
        pltpu.VMEM((_K, _DW), jnp.float32),
        pltpu.SemaphoreType.DMA,
    ],
)
def _deg_kernel(dst_hbm, ones_hbm, zeros_hbm, out_hbm, acc, didx, ones_v, sem):
    cc = lax.axis_index("c")
    sid = lax.axis_index("s")
    wid = sid * _NC + cc
    pltpu.sync_copy(ones_hbm, ones_v)
    pltpu.sync_copy(dst_hbm.at[wid], didx)
    pltpu.sync_copy(zeros_hbm, acc.at[pl.ds(sid * _DPT, _DPT)])
    plsc.subcore_barrier()

    def block(bi, carry):
        def fire(ci, carry2):
            pltpu.async_copy(ones_v, acc.at[didx.at[bi * 25 + ci]], sem,
                             add=True)
            return carry2

        lax.fori_loop(0, 25, fire, 0)

        def drain(ci, carry2):
            pltpu.make_async_copy(ones_v, acc.at[didx.at[0]], sem).wait()
            return carry2

        lax.fori_loop(0, 25, drain, 0)
        return carry

    lax.fori_loop(0, _CH // 25, block, 0)
    plsc.subcore_barrier()
    pltpu.sync_copy(
        acc.at[pl.ds(sid * _DPT, _DPT)],
        out_hbm.at[cc, pl.ds(sid * _DPT, _DPT)],
    )


@functools.partial(
    pl.kernel,
    out_type=jax.ShapeDtypeStruct((_NC, _NR, _D), jnp.float32),
    mesh=_mesh,
    scratch_types=[
        pltpu.VMEM_SHARED((_NR, _D), jnp.float32),
        pltpu.VMEM((_ECS, _EK), jnp.int32),
        pltpu.VMEM((_ECS, _EK), jnp.int32),
        pltpu.VMEM((_EK, _D), jnp.float32),
        pltpu.VMEM((_EK, _D), jnp.float32),
        pltpu.SemaphoreType.DMA,
        pltpu.SemaphoreType.DMA,
        pltpu.SemaphoreType.DMA,
        pltpu.SemaphoreType.DMA,
    ],
)
def _edge_kernel(src_hbm, dst_hbm, zeros_hbm, g_hbm, out_hbm,
                 acc, sidx, didx, rows_a, rows_b, sem_ga, sem_gb, sem_sa, sem_sb):
    cc = lax.axis_index("c")
    sid = lax.axis_index("s")
    wid = sid * _NC + cc
    pltpu.sync_copy(zeros_hbm, acc.at[pl.ds(sid * _RPT, _RPT)])
    plsc.subcore_barrier()

    def wait_g(sem, buf):
        pltpu.make_async_copy(g_hbm.at[sidx.at[0]], buf, sem).wait()

    def wait_s(sem, buf):
        pltpu.make_async_copy(buf, acc.at[didx.at[0]], sem).wait()

    def stage(si, carry):
        pltpu.sync_copy(src_hbm.at[wid, si], sidx)
        pltpu.sync_copy(dst_hbm.at[wid, si], didx)
        pltpu.async_copy(g_hbm.at[sidx.at[0]], rows_a, sem_ga)
        pltpu.async_copy(g_hbm.at[sidx.at[1]], rows_b, sem_gb)

        def body(p, carry2):
            i = 2 * p
            wait_g(sem_ga, rows_a)
            pltpu.async_copy(rows_a, acc.at[didx.at[i]], sem_sa, add=True)
            wait_g(sem_gb, rows_b)
            pltpu.async_copy(rows_b, acc.at[didx.at[i + 1]], sem_sb, add=True)
            wait_s(sem_sa, rows_a)
            pltpu.async_copy(g_hbm.at[sidx.at[i + 2]], rows_a, sem_ga)
            wait_s(sem_sb, rows_b)
            pltpu.async_copy(g_hbm.at[sidx.at[i + 3]], rows_b, sem_gb)
            return carry2

        lax.fori_loop(0, _ECS // 2 - 1, body, 0)
        wait_g(sem_ga, rows_a)
        pltpu.async_copy(rows_a, acc.at[didx.at[_ECS - 2]], sem_sa, add=True)
        wait_g(sem_gb, rows_b)
        pltpu.async_copy(rows_b, acc.at[didx.at[_ECS - 1]], sem_sb, add=True)
        wait_s(sem_sa, rows_a)
        wait_s(sem_sb, rows_b)
        return carry

    lax.fori_loop(0, _ENST, stage, 0)
    plsc.subcore_barrier()
    pltpu.sync_copy(
        acc.at[pl.ds(sid * _RPT, _RPT)],
        out_hbm.at[cc, pl.ds(sid * _RPT, _RPT)],
    )


def _dinv(deg_ref):
    return lax.rsqrt(deg_ref[0, :_N, :] + deg_ref[1, :_N, :] + 1.0)


def _mm1_body(x_ref, w_ref, deg_ref, out_ref):
    hw = jnp.dot(x_ref[...], w_ref[...], preferred_element_type=jnp.float32)
    out_ref[...] = hw * _dinv(deg_ref)


def _mid_body(s_ref, g_ref, deg_ref, b_ref, w_ref, out_ref):
    dinv = _dinv(deg_ref)
    h = (s_ref[0, :_N, :] + s_ref[1, :_N, :] + g_ref[...]) * dinv + b_ref[...]
    h = jnp.maximum(h, 0.0)
    hw = jnp.dot(h, w_ref[...], preferred_element_type=jnp.float32)
    out_ref[...] = hw * dinv


def _final_body(s_ref, g_ref, deg_ref, b_ref, out_ref):
    dinv = _dinv(deg_ref)
    h = (s_ref[0, :_N, :] + s_ref[1, :_N, :] + g_ref[...]) * dinv + b_ref[...]
    h = jnp.maximum(h, 0.0)
    nrm = jnp.sqrt(jnp.sum(h * h, axis=1, keepdims=True))
    out_ref[...] = h / (nrm + 1e-12)


_f32_out = jax.ShapeDtypeStruct((_N, _D), jnp.float32)
_mm1 = pl.pallas_call(_mm1_body, out_shape=_f32_out)
_mid = pl.pallas_call(_mid_body, out_shape=_f32_out)
_final = pl.pallas_call(_final_body, out_shape=_f32_out)


@jax.jit
def _impl(x, edge_index, W0, b0, W1, b1):
    src = edge_index[0].astype(jnp.int32)
    dst = edge_index[1].astype(jnp.int32)
    dst3 = dst.reshape(_NW, _CH, _K)
    src4 = src.reshape(_NW, _ENST, _ECS, _EK)
    dst4 = dst.reshape(_NW, _ENST, _ECS, _EK)
    ones = jnp.ones((_K, _DW), jnp.float32)
    dzeros = jnp.zeros((_DPT, _DW), jnp.float32)
    rzeros = jnp.zeros((_RPT, _D), jnp.float32)

    degp = _deg_kernel(dst3, ones, dzeros)
    b0r = b0.reshape(1, _D)
    b1r = b1.reshape(1, _D)

    g0 = _mm1(x, W0, degp)
    s0 = _edge_kernel(src4, dst4, rzeros, g0)
    g1 = _mid(s0, g0, degp, b0r, W1)
    s1 = _edge_kernel(src4, dst4, rzeros, g1)
    return _final(s1, g1, degp, b1r)


def kernel(x, edge_index, W0, b0, W1, b1):
    return _impl(x, edge_index, W0, b0, W1, b1)


# speedup vs baseline: 22.5760x; 1.0223x over previous
import functools

import jax
import jax.numpy as jnp
from jax import lax
from jax.experimental import pallas as pl
from jax.experimental.pallas import tpu as pltpu
from jax.experimental.pallas import tpu_sc as plsc

_N = 10000
_D = 128
_E = 320000

_NC = 2
_NS = 16
_NW = _NC * _NS
_EPW = _E // _NW
_K = 80
_CH = _EPW // _K

_EK = 100
_ENST = 2
_ECS = _EPW // (_EK * _ENST)
_NR = 10240
_RPT = _NR // _NS
_ZR = 128

_NP = 10240
_DW = 128
_DPT = _NP // _NS

_mesh = plsc.VectorSubcoreMesh(
    core_axis_name="c", subcore_axis_name="s", num_cores=_NC, num_subcores=_NS
)


@functools.partial(
    pl.kernel,
    out_type=jax.ShapeDtypeStruct((_NC, _NP, _DW), jnp.float32),
    mesh=_mesh,
    scratch_types=[
        pltpu.VMEM_SHARED((_NP, _DW), jnp.float32),
        pltpu.VMEM((_CH, _K), jnp.int32),
        pltpu.VMEM((_K, _DW), jnp.float32),
        pltpu.SemaphoreType.DMA,
    ],
)
def _deg_kernel(dst_hbm, ones_hbm, zeros_hbm, out_hbm, acc, didx, ones_v, sem):
    cc = lax.axis_index("c")
    sid = lax.axis_index("s")
    wid = sid * _NC + cc
    pltpu.sync_copy(ones_hbm, ones_v)
    pltpu.sync_copy(dst_hbm.at[wid], didx)
    pltpu.sync_copy(zeros_hbm, acc.at[pl.ds(sid * _DPT, _DPT)])
    plsc.subcore_barrier()

    def block(bi, carry):
        def fire(ci, carry2):
            pltpu.async_copy(ones_v, acc.at[didx.at[bi * 25 + ci]], sem,
                             add=True)
            return carry2

        lax.fori_loop(0, 25, fire, 0)

        def drain(ci, carry2):
            pltpu.make_async_copy(ones_v, acc.at[didx.at[0]], sem).wait()
            return carry2

        lax.fori_loop(0, 25, drain, 0)
        return carry

    lax.fori_loop(0, _CH // 25, block, 0)
    plsc.subcore_barrier()
    pltpu.sync_copy(
        acc.at[pl.ds(sid * _DPT, _DPT)],
        out_hbm.at[cc, pl.ds(sid * _DPT, _DPT)],
    )


@functools.partial(
    pl.kernel,
    out_type=jax.ShapeDtypeStruct((_NC, _NR, _D), jnp.float32),
    mesh=_mesh,
    scratch_types=[
        pltpu.VMEM_SHARED((_NR, _D), jnp.float32),
        pltpu.VMEM((_ECS, _EK), jnp.int32),
        pltpu.VMEM((_ECS, _EK), jnp.int32),
        pltpu.VMEM((_EK, _D), jnp.float32),
        pltpu.VMEM((_EK, _D), jnp.float32),
        pltpu.SemaphoreType.DMA,
        pltpu.SemaphoreType.DMA,
        pltpu.SemaphoreType.DMA,
        pltpu.SemaphoreType.DMA,
    ],
)
def _edge_kernel(src_hbm, dst_hbm, zeros_hbm, g_hbm, out_hbm,
                 acc, sidx, didx, rows_a, rows_b, sem_ga, sem_gb, sem_sa, sem_sb):
    cc = lax.axis_index("c")
    sid = lax.axis_index("s")
    wid = sid * _NC + cc
    pltpu.sync_copy(zeros_hbm, acc.at[pl.ds(sid * _RPT, _RPT)])
    plsc.subcore_barrier()

    def wait_g(sem, buf):
        pltpu.make_async_copy(g_hbm.at[sidx.at[0]], buf, sem).wait()

    def wait_s(sem, buf):
        pltpu.make_async_copy(buf, acc.at[didx.at[0]], sem).wait()

    def stage(si, carry):
        pltpu.sync_copy(src_hbm.at[wid, si], sidx)
        pltpu.sync_copy(dst_hbm.at[wid, si], didx)
        pltpu.async_copy(g_hbm.at[sidx.at[0]], rows_a, sem_ga)
        pltpu.async_copy(g_hbm.at[sidx.at[1]], rows_b, sem_gb)

        def body(p, carry2):
            i = 2 * p
            wait_g(sem_ga, rows_a)
            pltpu.async_copy(rows_a, acc.at[didx.at[i]], sem_sa, add=True)
            wait_g(sem_gb, rows_b)
            pltpu.async_copy(rows_b, acc.at[didx.at[i + 1]], sem_sb, add=True)
            wait_s(sem_sa, rows_a)
            pltpu.async_copy(g_hbm.at[sidx.at[i + 2]], rows_a, sem_ga)
            wait_s(sem_sb, rows_b)
            pltpu.async_copy(g_hbm.at[sidx.at[i + 3]], rows_b, sem_gb)
            return carry2

        lax.fori_loop(0, _ECS // 2 - 1, body, 0)
        wait_g(sem_ga, rows_a)
        pltpu.async_copy(rows_a, acc.at[didx.at[_ECS - 2]], sem_sa, add=True)
        wait_g(sem_gb, rows_b)
        pltpu.async_copy(rows_b, acc.at[didx.at[_ECS - 1]], sem_sb, add=True)
        wait_s(sem_sa, rows_a)
        wait_s(sem_sb, rows_b)
        return carry

    lax.fori_loop(0, _ENST, stage, 0)
    plsc.subcore_barrier()
    pltpu.sync_copy(
        acc.at[pl.ds(sid * _RPT, _RPT)],
        out_hbm.at[cc, pl.ds(sid * _RPT, _RPT)],
    )


def _dinv(deg_ref):
    return lax.rsqrt(deg_ref[0, :_N, :] + deg_ref[1, :_N, :] + 1.0)


def _mm1_body(x_ref, w_ref, deg_ref, out_ref):
    hw = jnp.dot(x_ref[...], w_ref[...], preferred_element_type=jnp.float32)
    out_ref[...] = hw * _dinv(deg_ref)


def _mid_body(s_ref, g_ref, deg_ref, b_ref, w_ref, out_ref):
    dinv = _dinv(deg_ref)
    h = (s_ref[0, :_N, :] + s_ref[1, :_N, :] + g_ref[...]) * dinv + b_ref[...]
    h = jnp.maximum(h, 0.0)
    hw = jnp.dot(h, w_ref[...], preferred_element_type=jnp.float32)
    out_ref[...] = hw * dinv


def _final_body(s_ref, g_ref, deg_ref, b_ref, out_ref):
    dinv = _dinv(deg_ref)
    h = (s_ref[0, :_N, :] + s_ref[1, :_N, :] + g_ref[...]) * dinv + b_ref[...]
    h = jnp.maximum(h, 0.0)
    nrm = jnp.sqrt(jnp.sum(h * h, axis=1, keepdims=True))
    out_ref[...] = h / (nrm + 1e-12)


_f32_out = jax.ShapeDtypeStruct((_N, _D), jnp.float32)
_mm1 = pl.pallas_call(_mm1_body, out_shape=_f32_out)
_mid = pl.pallas_call(_mid_body, out_shape=_f32_out)
_final = pl.pallas_call(_final_body, out_shape=_f32_out)


@jax.jit
def _impl(x, edge_index, W0, b0, W1, b1):
    src = edge_index[0].astype(jnp.int32)
    dst = edge_index[1].astype(jnp.int32)
    dst3 = dst.reshape(_NW, _CH, _K)
    src4 = src.reshape(_NW, _ENST, _ECS, _EK)
    dst4 = dst.reshape(_NW, _ENST, _ECS, _EK)
    ones = jnp.ones((_K, _DW), jnp.float32)
    dzeros = jnp.zeros((_DPT, _DW), jnp.float32)
    rzeros = jnp.zeros((_RPT, _D), jnp.float32)

    degp = _deg_kernel(dst3, ones, dzeros)
    b0r = b0.reshape(1, _D)
    b1r = b1.reshape(1, _D)

    g0 = _mm1(x, W0, degp)
    s0 = _edge_kernel(src4, dst4, rzeros, g0)
    g1 = _mid(s0, g0, degp, b0r, W1)
    s1 = _edge_kernel(src4, dst4, rzeros, g1)
    return _final(s1, g1, degp, b1r)


def kernel(x, edge_index, W0, b0, W1, b1):
    return _impl(x, edge_index, W0, b0, W1, b1)
